# Initial kernel scaffold; baseline (speedup 1.0000x reference)
import functools

import jax
import jax.numpy as jnp
from jax import lax
from jax.experimental import pallas as pl
from jax.experimental.pallas import tpu as pltpu
from jax.experimental.pallas import tpu_sc as plsc

D = 128
NC, NS = 2, 16
NW = NC * NS
CHUNK = 128
ROWS_PER_TILE = 626
ROWS_PAD = NS * ROWS_PER_TILE


def _sc_scatter(src_p, dst_p, H):
    n_chunks = src_p.shape[1]
    mesh = plsc.VectorSubcoreMesh(core_axis_name="c", subcore_axis_name="s")

    @functools.partial(
        pl.kernel,
        out_type=jax.ShapeDtypeStruct((NC * ROWS_PAD, D), jnp.float32),
        mesh=mesh,
        scratch_types=[
            pltpu.VMEM((n_chunks, CHUNK), jnp.int32),
            pltpu.VMEM((n_chunks, CHUNK), jnp.int32),
            pltpu.VMEM((CHUNK, D), jnp.float32),
            pltpu.VMEM_SHARED((ROWS_PAD, D), jnp.float32),
        ],
    )
    def k(src_hbm, dst_hbm, h_hbm, out_hbm, src_v, dst_v, rows, acc):
        cid = lax.axis_index("c")
        sid = lax.axis_index("s")
        wid = sid * NC + cid

        pltpu.sync_copy(src_hbm.at[wid], src_v)
        pltpu.sync_copy(dst_hbm.at[wid], dst_v)

        zeros = jnp.zeros((16,), jnp.float32)

        def zbody(i, carry):
            for j in range(D // 16):
                rows[i, pl.ds(j * 16, 16)] = zeros
            return carry

        lax.fori_loop(0, CHUNK, zbody, 0)
        base = sid * ROWS_PER_TILE
        off = 0
        while off < ROWS_PER_TILE:
            n = min(CHUNK, ROWS_PER_TILE - off)
            pltpu.sync_copy(rows.at[pl.ds(0, n)], acc.at[pl.ds(base + off, n)])
            off += n
        plsc.subcore_barrier()

        def loop_body(c, carry):
            pltpu.sync_copy(h_hbm.at[src_v.at[c]], rows)
            pltpu.sync_copy(rows, acc.at[dst_v.at[c]], add=True)
            return carry

        lax.fori_loop(0, n_chunks, loop_body, 0)
        plsc.subcore_barrier()

        out_base = cid * ROWS_PAD + sid * ROWS_PER_TILE
        pltpu.sync_copy(
            acc.at[pl.ds(sid * ROWS_PER_TILE, ROWS_PER_TILE)],
            out_hbm.at[pl.ds(out_base, ROWS_PER_TILE)],
        )

    return k(src_p, dst_p, H)


def _tc_matmul(H, P0, P1, W):
    n_nodes = H.shape[0]
    bm = 1000

    def body(h_ref, p0_ref, p1_ref, w_ref, o_ref):
        x = h_ref[...] + p0_ref[...] + p1_ref[...]
        y = jnp.dot(x, w_ref[...], preferred_element_type=jnp.float32)
        o_ref[...] = jnp.maximum(y, 0.0)

    return pl.pallas_call(
        body,
        grid=(n_nodes // bm,),
        in_specs=[
            pl.BlockSpec((bm, D), lambda i: (i, 0)),
            pl.BlockSpec((bm, D), lambda i: (i, 0)),
            pl.BlockSpec((bm, D), lambda i: (i, 0)),
            pl.BlockSpec((D, D), lambda i: (0, 0)),
        ],
        out_specs=pl.BlockSpec((bm, D), lambda i: (i, 0)),
        out_shape=jax.ShapeDtypeStruct((n_nodes, D), jnp.float32),
    )(H, P0, P1, W)


def kernel(H, edge_index, W):
    n_nodes = H.shape[0]
    dump = n_nodes
    src = edge_index[0].astype(jnp.int32)
    dst = edge_index[1].astype(jnp.int32)
    e = src.shape[0]

    per_round = NW * CHUNK
    n_chunks = -(-e // per_round)
    if n_chunks % 2:
        n_chunks += 1
    total = NW * n_chunks * CHUNK
    pad = total - e
    src_p = jnp.concatenate([src, jnp.zeros((pad,), jnp.int32)]).reshape(
        NW, n_chunks, CHUNK
    )
    dst_p = jnp.concatenate([dst, jnp.full((pad,), dump, jnp.int32)]).reshape(
        NW, n_chunks, CHUNK
    )

    partials = _sc_scatter(src_p, dst_p, H)
    p0 = partials[:n_nodes]
    p1 = partials[ROWS_PAD : ROWS_PAD + n_nodes]
    return _tc_matmul(H, p0, p1, W)


# speedup vs baseline: 2.8693x; 2.8693x over previous
import functools

import jax
import jax.numpy as jnp
from jax import lax
from jax.experimental import pallas as pl
from jax.experimental.pallas import tpu as pltpu
from jax.experimental.pallas import tpu_sc as plsc

D = 128
NC, NS = 2, 16
NW = NC * NS
CHUNK = 128
ROWS_PER_TILE = 632
ROWS_PAD = NS * ROWS_PER_TILE


def _sc_scatter(src_p, dst_p, H):
    n_chunks = src_p.shape[1]
    mesh = plsc.VectorSubcoreMesh(core_axis_name="c", subcore_axis_name="s")

    @functools.partial(
        pl.kernel,
        out_type=jax.ShapeDtypeStruct((NC * ROWS_PAD, D), jnp.float32),
        mesh=mesh,
        scratch_types=[
            pltpu.VMEM((n_chunks, CHUNK), jnp.int32),
            pltpu.VMEM((n_chunks, CHUNK), jnp.int32),
            pltpu.VMEM((CHUNK, D), jnp.float32),
            pltpu.VMEM_SHARED((ROWS_PAD, D), jnp.float32),
        ],
    )
    def k(src_hbm, dst_hbm, h_hbm, out_hbm, src_v, dst_v, rows, acc):
        cid = lax.axis_index("c")
        sid = lax.axis_index("s")
        wid = sid * NC + cid

        pltpu.sync_copy(src_hbm.at[wid], src_v)
        pltpu.sync_copy(dst_hbm.at[wid], dst_v)

        zeros = jnp.zeros((16,), jnp.float32)

        def zbody(i, carry):
            for j in range(D // 16):
                rows[i, pl.ds(j * 16, 16)] = zeros
            return carry

        lax.fori_loop(0, CHUNK, zbody, 0)
        base = sid * ROWS_PER_TILE
        off = 0
        while off < ROWS_PER_TILE:
            n = min(CHUNK, ROWS_PER_TILE - off)
            pltpu.sync_copy(rows.at[pl.ds(0, n)], acc.at[pl.ds(base + off, n)])
            off += n
        plsc.subcore_barrier()

        def loop_body(c, carry):
            pltpu.sync_copy(h_hbm.at[src_v.at[c]], rows)
            pltpu.sync_copy(rows, acc.at[dst_v.at[c]], add=True)
            return carry

        lax.fori_loop(0, n_chunks, loop_body, 0)
        plsc.subcore_barrier()

        out_base = cid * ROWS_PAD + sid * ROWS_PER_TILE
        pltpu.sync_copy(
            acc.at[pl.ds(sid * ROWS_PER_TILE, ROWS_PER_TILE)],
            out_hbm.at[pl.ds(out_base, ROWS_PER_TILE)],
        )

    return k(src_p, dst_p, H)


def _tc_matmul(H, P0, P1, W):
    n_nodes = H.shape[0]
    bm = 1000

    def body(h_ref, p0_ref, p1_ref, w_ref, o_ref):
        x = h_ref[...] + p0_ref[...] + p1_ref[...]
        y = jnp.dot(x, w_ref[...], preferred_element_type=jnp.float32)
        o_ref[...] = jnp.maximum(y, 0.0)

    return pl.pallas_call(
        body,
        grid=(n_nodes // bm,),
        in_specs=[
            pl.BlockSpec((bm, D), lambda i: (i, 0)),
            pl.BlockSpec((bm, D), lambda i: (i, 0)),
            pl.BlockSpec((bm, D), lambda i: (i, 0)),
            pl.BlockSpec((D, D), lambda i: (0, 0)),
        ],
        out_specs=pl.BlockSpec((bm, D), lambda i: (i, 0)),
        out_shape=jax.ShapeDtypeStruct((n_nodes, D), jnp.float32),
    )(H, P0, P1, W)


def kernel(H, edge_index, W):
    n_nodes = H.shape[0]
    dump = n_nodes
    src = edge_index[0].astype(jnp.int32)
    dst = edge_index[1].astype(jnp.int32)
    e = src.shape[0]

    per_round = NW * CHUNK
    n_chunks = -(-e // per_round)
    if n_chunks % 2:
        n_chunks += 1
    total = NW * n_chunks * CHUNK
    pad = total - e
    src_p = jnp.concatenate([src, jnp.zeros((pad,), jnp.int32)]).reshape(
        NW, n_chunks, CHUNK
    )
    dst_p = jnp.concatenate([dst, jnp.full((pad,), dump, jnp.int32)]).reshape(
        NW, n_chunks, CHUNK
    )

    partials = _sc_scatter(src_p, dst_p, H)
    p0 = partials[:n_nodes]
    p1 = partials[ROWS_PAD : ROWS_PAD + n_nodes]
    return _tc_matmul(H, p0, p1, W)


# speedup vs baseline: 4.6239x; 1.6115x over previous
import functools

import jax
import jax.numpy as jnp
from jax import lax
from jax.experimental import pallas as pl
from jax.experimental.pallas import tpu as pltpu
from jax.experimental.pallas import tpu_sc as plsc

D = 128
DH = D // 2
NC, NS = 2, 16
CHUNK = 128
ROWS_PER_TILE = 632
ROWS_PAD = NS * ROWS_PER_TILE
NBUF = 4
LOOK = 2


def _sc_scatter(src_p, dst_p, Hs):
    n_chunks = src_p.shape[1]
    assert n_chunks % NBUF == 0 and n_chunks >= 2 * NBUF
    n_groups = n_chunks // NBUF
    mesh = plsc.VectorSubcoreMesh(core_axis_name="c", subcore_axis_name="s")

    @functools.partial(
        pl.kernel,
        out_type=jax.ShapeDtypeStruct((NC, ROWS_PAD, DH), jnp.float32),
        mesh=mesh,
        scratch_types=[
            pltpu.VMEM((n_chunks, CHUNK), jnp.int32),
            pltpu.VMEM((n_chunks, CHUNK), jnp.int32),
            [pltpu.VMEM((CHUNK, DH), jnp.float32) for _ in range(NBUF)],
            pltpu.VMEM_SHARED((ROWS_PAD, DH), jnp.float32),
            [pltpu.SemaphoreType.DMA for _ in range(NBUF)],
            [pltpu.SemaphoreType.DMA for _ in range(NBUF)],
        ],
        compiler_params=pltpu.CompilerParams(use_tc_tiling_on_sc=False),
    )
    def k(src_hbm, dst_hbm, hs_hbm, out_hbm, src_v, dst_v, rows, acc, gsem, ssem):
        cid = lax.axis_index("c")
        sid = lax.axis_index("s")
        h_half = hs_hbm.at[cid]

        pltpu.sync_copy(src_hbm.at[sid], src_v)
        pltpu.sync_copy(dst_hbm.at[sid], dst_v)

        zeros = jnp.zeros((16,), jnp.float32)

        def zbody(i, carry):
            for j in range(DH // 16):
                rows[0][i, pl.ds(j * 16, 16)] = zeros
            return carry

        lax.fori_loop(0, CHUNK, zbody, 0)
        base = sid * ROWS_PER_TILE
        off = 0
        while off < ROWS_PER_TILE:
            n = min(CHUNK, ROWS_PER_TILE - off)
            pltpu.sync_copy(rows[0].at[pl.ds(0, n)], acc.at[pl.ds(base + off, n)])
            off += n
        plsc.subcore_barrier()

        def issue_gather(c, b):
            pltpu.async_copy(h_half.at[src_v.at[c]], rows[b], gsem[b])

        def wait_gather(b):
            pltpu.make_async_copy(h_half.at[src_v.at[0]], rows[b], gsem[b]).wait()

        def issue_scatter(c, b):
            pltpu.async_copy(rows[b], acc.at[dst_v.at[c]], ssem[b], add=True)

        def wait_scatter(b):
            pltpu.make_async_copy(rows[b], acc.at[dst_v.at[0]], ssem[b]).wait()

        def step(c, b, first, last):
            bf = (b + LOOK) % NBUF
            do_gather = not last or b < NBUF - LOOK
            if do_gather:
                if not (first and b < LOOK):
                    wait_scatter(bf)
                issue_gather(c + LOOK, bf)
            wait_gather(b)
            issue_scatter(c, b)

        for b in range(LOOK):
            issue_gather(b, b)
        for b in range(NBUF):
            step(b, b, first=True, last=False)

        def gbody(g, carry):
            c0 = g * NBUF
            for b in range(NBUF):
                step(c0 + b, b, first=False, last=False)
            return carry

        lax.fori_loop(1, n_groups - 1, gbody, 0)
        c0 = (n_groups - 1) * NBUF
        for b in range(NBUF):
            step(c0 + b, b, first=False, last=True)
        for b in range(NBUF):
            wait_scatter(b)
        plsc.subcore_barrier()

        pltpu.sync_copy(
            acc.at[pl.ds(sid * ROWS_PER_TILE, ROWS_PER_TILE)],
            out_hbm.at[cid].at[pl.ds(sid * ROWS_PER_TILE, ROWS_PER_TILE)],
        )

    return k(src_p, dst_p, Hs)


def _tc_matmul(H, NH0, NH1, W):
    n_nodes = H.shape[0]
    bm = 1000

    def body(h_ref, n0_ref, n1_ref, w_ref, o_ref):
        x = h_ref[...] + jnp.concatenate([n0_ref[...], n1_ref[...]], axis=1)
        y = jnp.dot(x, w_ref[...], preferred_element_type=jnp.float32)
        o_ref[...] = jnp.maximum(y, 0.0)

    return pl.pallas_call(
        body,
        grid=(n_nodes // bm,),
        in_specs=[
            pl.BlockSpec((bm, D), lambda i: (i, 0)),
            pl.BlockSpec((bm, DH), lambda i: (i, 0)),
            pl.BlockSpec((bm, DH), lambda i: (i, 0)),
            pl.BlockSpec((D, D), lambda i: (0, 0)),
        ],
        out_specs=pl.BlockSpec((bm, D), lambda i: (i, 0)),
        out_shape=jax.ShapeDtypeStruct((n_nodes, D), jnp.float32),
    )(H, NH0, NH1, W)


def kernel(H, edge_index, W):
    n_nodes = H.shape[0]
    dump = n_nodes
    src = edge_index[0].astype(jnp.int32)
    dst = edge_index[1].astype(jnp.int32)
    e = src.shape[0]

    per_round = NS * CHUNK
    n_chunks = -(-e // per_round)
    n_chunks += (-n_chunks) % NBUF
    total = NS * n_chunks * CHUNK
    pad = total - e
    src_p = jnp.concatenate([src, jnp.zeros((pad,), jnp.int32)]).reshape(
        NS, n_chunks, CHUNK
    )
    dst_p = jnp.concatenate([dst, jnp.full((pad,), dump, jnp.int32)]).reshape(
        NS, n_chunks, CHUNK
    )
    Hs = jnp.moveaxis(H.reshape(n_nodes, NC, DH), 1, 0)

    parts = _sc_scatter(src_p, dst_p, Hs)
    nh0 = parts[0, :n_nodes]
    nh1 = parts[1, :n_nodes]
    return _tc_matmul(H, nh0, nh1, W)
